# baseline (speedup 1.0000x reference)
import functools

import jax
import jax.numpy as jnp
from jax import lax
from jax.experimental import pallas as pl
from jax.experimental.pallas import tpu as pltpu
from jax.experimental.pallas import tpu_sc as plsc

NC, NS = 2, 16
NW = NC * NS
CHUNK = 128
NSLICES = 4
ROWS_BLK = 2048


@functools.partial(jax.jit, static_argnums=(2, 3, 4))
def _sc_gather(labels3, emb_table, n_chunks, n_rows, emb_d):
    mesh = plsc.VectorSubcoreMesh(
        core_axis_name="c", subcore_axis_name="s",
        num_cores=NC, num_subcores=NS,
    )
    L = 8
    G = 4
    assert n_chunks > L

    @functools.partial(
        pl.kernel,
        out_type=jax.ShapeDtypeStruct((NW * n_chunks * CHUNK, emb_d), jnp.float32),
        mesh=mesh,
        scratch_types=[
            pltpu.VMEM((n_chunks, CHUNK), jnp.int32),
            pltpu.VMEM((L, CHUNK, emb_d), jnp.float32),
            pltpu.SemaphoreType.DMA((L,)),
            pltpu.SemaphoreType.DMA((L,)),
        ],
        compiler_params=pltpu.CompilerParams(use_tc_tiling_on_sc=False),
    )
    def gather_kernel(labels_hbm, table2d, out_hbm, idx_v, rows_v, gsem, wsem):
        wid = lax.axis_index("c") * NS + lax.axis_index("s")
        base = wid * (n_chunks * CHUNK)
        pltpu.sync_copy(labels_hbm.at[wid], idx_v)

        for k in range(G):
            pltpu.async_copy(
                table2d.at[idx_v.at[k]], rows_v.at[k % L], gsem.at[k % L])

        def body(j, carry):
            slot = lax.rem(j, L)
            pltpu.make_async_copy(
                table2d.at[idx_v.at[j]], rows_v.at[slot], gsem.at[slot]).wait()
            pltpu.async_copy(
                rows_v.at[slot],
                out_hbm.at[pl.ds(base + j * CHUNK, CHUNK)],
                wsem.at[slot])

            nslot = lax.rem(j + G, L)

            @pl.when(j + G < n_chunks)
            def _issue_next():
                @pl.when(j + G >= L)
                def _drain_write():
                    pltpu.make_async_copy(
                        rows_v.at[nslot],
                        out_hbm.at[pl.ds(base, CHUNK)],
                        wsem.at[nslot]).wait()

                pltpu.async_copy(
                    table2d.at[idx_v.at[j + G]], rows_v.at[nslot],
                    gsem.at[nslot])

            return carry

        lax.fori_loop(0, n_chunks, body, 0)

        for j in range(max(0, n_chunks - L), n_chunks):
            pltpu.make_async_copy(
                rows_v.at[j % L],
                out_hbm.at[pl.ds(base, CHUNK)],
                wsem.at[j % L]).wait()

    return gather_kernel(labels3, emb_table)


def _mm_first_body(v_ref, e_ref, wv_ref, we_ref, b_ref, o_ref):
    acc = jnp.dot(v_ref[...], wv_ref[...], preferred_element_type=jnp.float32)
    acc += jnp.dot(e_ref[...], we_ref[...], preferred_element_type=jnp.float32)
    o_ref[...] = acc + b_ref[...]


def _mm_chain_body(prev_ref, v_ref, e_ref, wv_ref, we_ref, b_ref, o_ref):
    del prev_ref
    _mm_first_body(v_ref, e_ref, wv_ref, we_ref, b_ref, o_ref)


@functools.partial(jax.jit, static_argnums=(6, 7))
def _tc_project_slice(prev, values_flat, emb_s, Wv, We, b2, s, n_total):
    Q, ED = emb_s.shape
    VD = values_flat.shape[1]
    LD = Wv.shape[1]
    R = ROWS_BLK
    blk_off = s * (Q // R)
    grid = (Q // R,)

    common_in_specs = [
        pl.BlockSpec((R, VD), lambda i: (blk_off + i, 0)),
        pl.BlockSpec((R, ED), lambda i: (i, 0)),
        pl.BlockSpec((VD, LD), lambda i: (0, 0)),
        pl.BlockSpec((ED, LD), lambda i: (0, 0)),
        pl.BlockSpec((1, LD), lambda i: (0, 0)),
    ]
    out_spec = pl.BlockSpec((R, LD), lambda i: (blk_off + i, 0))
    out_shape = jax.ShapeDtypeStruct((n_total, LD), jnp.float32)

    if prev is None:
        return pl.pallas_call(
            _mm_first_body,
            grid=grid,
            in_specs=common_in_specs,
            out_specs=out_spec,
            out_shape=out_shape,
        )(values_flat, emb_s, Wv, We, b2)
    return pl.pallas_call(
        _mm_chain_body,
        grid=grid,
        in_specs=[pl.BlockSpec(memory_space=pl.ANY)] + common_in_specs,
        out_specs=out_spec,
        out_shape=out_shape,
        input_output_aliases={0: 0},
    )(prev, values_flat, emb_s, Wv, We, b2)


def kernel(values, labels, emb_table, W, b):
    B, T, VD = values.shape
    ED = emb_table.shape[1]
    LD = W.shape[1]
    N = B * T
    Q = N // NSLICES

    labels_flat = labels.reshape(N).astype(jnp.int32)
    values_flat = values.reshape(N, VD)
    Wv = W[:VD]
    We = W[VD:VD + ED]
    b2 = b.reshape(1, LD)

    n_chunks = Q // (NW * CHUNK)
    embs = []
    for s in range(NSLICES):
        labels3 = lax.dynamic_slice_in_dim(labels_flat, s * Q, Q).reshape(
            NW, n_chunks, CHUNK)
        embs.append(_sc_gather(labels3, emb_table, n_chunks,
                               emb_table.shape[0], ED))

    out = None
    for s in range(NSLICES):
        out = _tc_project_slice(out, values_flat, embs[s], Wv, We, b2, s, N)
    return out.reshape(B, T, LD)


# speedup vs baseline: 1.2161x; 1.2161x over previous
import functools

import jax
import jax.numpy as jnp
from jax import lax
from jax.experimental import pallas as pl
from jax.experimental.pallas import tpu as pltpu
from jax.experimental.pallas import tpu_sc as plsc

NC, NS = 2, 16
NW = NC * NS
CHUNK = 128
NSLICES = 4
ROWS_BLK = 2048


@functools.partial(jax.jit, static_argnums=(2, 3, 4))
def _sc_gather(labels3, emb_table, n_chunks, n_rows, emb_d):
    mesh = plsc.VectorSubcoreMesh(
        core_axis_name="c", subcore_axis_name="s",
        num_cores=NC, num_subcores=NS,
    )
    L = 8
    G = 4
    assert n_chunks > L
    pack = 128 // emb_d
    lines = CHUNK // pack

    VREG = 16
    n_vreg = emb_d // VREG

    @functools.partial(
        pl.kernel,
        out_type=jax.ShapeDtypeStruct(
            (NW * n_chunks * lines, pack * emb_d), jnp.float32),
        mesh=mesh,
        scratch_types=[
            pltpu.VMEM((n_chunks, CHUNK), jnp.int32),
            pltpu.VMEM((L, CHUNK, emb_d), jnp.float32),
            pltpu.VMEM((L, lines, pack * emb_d), jnp.float32),
            pltpu.SemaphoreType.DMA((L,)),
            pltpu.SemaphoreType.DMA((L,)),
        ],
        compiler_params=pltpu.CompilerParams(use_tc_tiling_on_sc=False),
    )
    def gather_kernel(labels_hbm, table2d, out_hbm, idx_v, rows_v, lines_v,
                      gsem, wsem):
        wid = lax.axis_index("c") * NS + lax.axis_index("s")
        base = wid * (n_chunks * lines)
        pltpu.sync_copy(labels_hbm.at[wid], idx_v)

        for k in range(G):
            pltpu.async_copy(
                table2d.at[idx_v.at[k]], rows_v.at[k % L], gsem.at[k % L])

        def body(j, carry):
            slot = lax.rem(j, L)
            pltpu.make_async_copy(
                table2d.at[idx_v.at[j]], rows_v.at[slot], gsem.at[slot]).wait()

            @pl.when(j >= L)
            def _drain_write():
                pltpu.make_async_copy(
                    lines_v.at[slot],
                    out_hbm.at[pl.ds(base, lines)],
                    wsem.at[slot]).wait()

            for m in range(lines):
                for k in range(pack):
                    for h in range(n_vreg):
                        lines_v[slot, m, pl.ds(k * emb_d + h * VREG, VREG)] = (
                            rows_v[slot, m * pack + k, pl.ds(h * VREG, VREG)])

            pltpu.async_copy(
                lines_v.at[slot],
                out_hbm.at[pl.ds(base + j * lines, lines)],
                wsem.at[slot])

            @pl.when(j + G < n_chunks)
            def _issue_next():
                pltpu.async_copy(
                    table2d.at[idx_v.at[j + G]], rows_v.at[lax.rem(j + G, L)],
                    gsem.at[lax.rem(j + G, L)])

            return carry

        lax.fori_loop(0, n_chunks, body, 0)

        for j in range(max(0, n_chunks - L), n_chunks):
            pltpu.make_async_copy(
                lines_v.at[j % L],
                out_hbm.at[pl.ds(base, lines)],
                wsem.at[j % L]).wait()

    return gather_kernel(labels3, emb_table)


def _mm_first_body(v_ref, e4_ref, wv_ref, we_ref, b_ref, o_ref):
    R = v_ref.shape[0]
    ED = we_ref.shape[0]
    LD = wv_ref.shape[1]
    e4 = e4_ref[...]
    pack = e4.shape[1] // ED
    cs = [
        jnp.dot(e4[:, k * ED:(k + 1) * ED], we_ref[...],
                preferred_element_type=jnp.float32)
        for k in range(pack)
    ]
    e_contrib = jnp.reshape(jnp.stack(cs, axis=1), (R, LD))
    acc = jnp.dot(v_ref[...], wv_ref[...], preferred_element_type=jnp.float32)
    o_ref[...] = acc + e_contrib + b_ref[...]


def _mm_chain_body(prev_ref, v_ref, e4_ref, wv_ref, we_ref, b_ref, o_ref):
    del prev_ref
    _mm_first_body(v_ref, e4_ref, wv_ref, we_ref, b_ref, o_ref)


@functools.partial(jax.jit, static_argnums=(6, 7))
def _tc_project_slice(prev, values_flat, emb4_s, Wv, We, b2, s, n_total):
    Q4, W128 = emb4_s.shape
    pack = W128 // We.shape[0]
    Q = Q4 * pack
    VD = values_flat.shape[1]
    ED = We.shape[0]
    LD = Wv.shape[1]
    R = ROWS_BLK
    blk_off = s * (Q // R)
    grid = (Q // R,)

    common_in_specs = [
        pl.BlockSpec((R, VD), lambda i: (blk_off + i, 0)),
        pl.BlockSpec((R // pack, W128), lambda i: (i, 0)),
        pl.BlockSpec((VD, LD), lambda i: (0, 0)),
        pl.BlockSpec((ED, LD), lambda i: (0, 0)),
        pl.BlockSpec((1, LD), lambda i: (0, 0)),
    ]
    out_spec = pl.BlockSpec((R, LD), lambda i: (blk_off + i, 0))
    out_shape = jax.ShapeDtypeStruct((n_total, LD), jnp.float32)

    if prev is None:
        return pl.pallas_call(
            _mm_first_body,
            grid=grid,
            in_specs=common_in_specs,
            out_specs=out_spec,
            out_shape=out_shape,
        )(values_flat, emb4_s, Wv, We, b2)
    return pl.pallas_call(
        _mm_chain_body,
        grid=grid,
        in_specs=[pl.BlockSpec(memory_space=pl.ANY)] + common_in_specs,
        out_specs=out_spec,
        out_shape=out_shape,
        input_output_aliases={0: 0},
    )(prev, values_flat, emb4_s, Wv, We, b2)


def kernel(values, labels, emb_table, W, b):
    B, T, VD = values.shape
    ED = emb_table.shape[1]
    LD = W.shape[1]
    N = B * T
    Q = N // NSLICES

    labels_flat = labels.reshape(N).astype(jnp.int32)
    values_flat = values.reshape(N, VD)
    Wv = W[:VD]
    We = W[VD:VD + ED]
    b2 = b.reshape(1, LD)

    n_chunks = Q // (NW * CHUNK)
    embs = []
    for s in range(NSLICES):
        labels3 = lax.dynamic_slice_in_dim(labels_flat, s * Q, Q).reshape(
            NW, n_chunks, CHUNK)
        embs.append(_sc_gather(labels3, emb_table, n_chunks,
                               emb_table.shape[0], ED))

    out = None
    for s in range(NSLICES):
        out = _tc_project_slice(out, values_flat, embs[s], Wv, We, b2, s, N)
    return out.reshape(B, T, LD)


# speedup vs baseline: 1.4339x; 1.1791x over previous
import functools

import jax
import jax.numpy as jnp
from jax import lax
from jax.experimental import pallas as pl
from jax.experimental.pallas import tpu as pltpu
from jax.experimental.pallas import tpu_sc as plsc

NC, NS = 2, 16
NW = NC * NS
CHUNK = 128
NSLICES = 5
ROWS_BLK = 2048


@functools.partial(jax.jit, static_argnums=(2, 3, 4))
def _sc_gather(labels3, emb_table, n_chunks, n_rows, emb_d):
    mesh = plsc.VectorSubcoreMesh(
        core_axis_name="c", subcore_axis_name="s",
        num_cores=NC, num_subcores=NS,
    )
    L = 8
    G = 4
    assert n_chunks > L
    pack = 128 // emb_d
    assert n_chunks % pack == 0
    lines_w = n_chunks * CHUNK // pack

    @functools.partial(
        pl.kernel,
        out_type=jax.ShapeDtypeStruct(
            (NW * lines_w, pack * emb_d), jnp.float32),
        mesh=mesh,
        scratch_types=[
            pltpu.VMEM((n_chunks, CHUNK), jnp.int32),
            pltpu.VMEM((L, CHUNK, emb_d), jnp.float32),
            pltpu.SemaphoreType.DMA((L,)),
            pltpu.SemaphoreType.DMA((L,)),
        ],
        compiler_params=pltpu.CompilerParams(use_tc_tiling_on_sc=False),
    )
    def gather_kernel(labels_hbm, table2d, out_hbm, idx_v, rows_v, gsem, wsem):
        wid = lax.axis_index("c") * NS + lax.axis_index("s")
        base = wid * lines_w
        pltpu.sync_copy(labels_hbm.at[wid], idx_v)

        for k in range(G):
            pltpu.async_copy(
                table2d.at[idx_v.at[k]], rows_v.at[k % L], gsem.at[k % L])

        def body(j, carry):
            slot = lax.rem(j, L)
            pltpu.make_async_copy(
                table2d.at[idx_v.at[j]], rows_v.at[slot], gsem.at[slot]).wait()
            line0 = base + lax.div(j, pack) * CHUNK
            lane0 = lax.rem(j, pack) * emb_d
            dst = out_hbm.at[pl.ds(line0, CHUNK), pl.ds(lane0, emb_d)]
            pltpu.async_copy(rows_v.at[slot], dst, wsem.at[slot])

            nslot = lax.rem(j + G, L)

            @pl.when(j + G < n_chunks)
            def _issue_next():
                @pl.when(j + G >= L)
                def _drain_write():
                    pltpu.make_async_copy(
                        rows_v.at[nslot],
                        out_hbm.at[pl.ds(base, CHUNK),
                                   pl.ds(0, emb_d)],
                        wsem.at[nslot]).wait()

                pltpu.async_copy(
                    table2d.at[idx_v.at[j + G]], rows_v.at[nslot],
                    gsem.at[nslot])

            return carry

        lax.fori_loop(0, n_chunks, body, 0)

        for j in range(max(0, n_chunks - L), n_chunks):
            pltpu.make_async_copy(
                rows_v.at[j % L],
                out_hbm.at[pl.ds(base, CHUNK), pl.ds(0, emb_d)],
                wsem.at[j % L]).wait()

    return gather_kernel(labels3, emb_table)


def _mm_first_body(v_ref, e4_ref, wv_ref, we_ref, b_ref, o_ref):
    R = v_ref.shape[0]
    ED = we_ref.shape[0]
    e4 = e4_ref[...]
    pack = e4.shape[1] // ED
    n_sub = R // (pack * CHUNK)
    cs = [
        jnp.dot(e4[:, k * ED:(k + 1) * ED], we_ref[...],
                preferred_element_type=jnp.float32)
        for k in range(pack)
    ]
    pieces = [
        cs[k][sb * CHUNK:(sb + 1) * CHUNK]
        for sb in range(n_sub)
        for k in range(pack)
    ]
    e_contrib = jnp.concatenate(pieces, axis=0)
    acc = jnp.dot(v_ref[...], wv_ref[...], preferred_element_type=jnp.float32)
    o_ref[...] = acc + e_contrib + b_ref[...]


def _mm_chain_body(prev_ref, v_ref, e4_ref, wv_ref, we_ref, b_ref, o_ref):
    del prev_ref
    _mm_first_body(v_ref, e4_ref, wv_ref, we_ref, b_ref, o_ref)


@functools.partial(jax.jit, static_argnums=(6, 7))
def _tc_project_slice(prev, values_flat, emb4_s, Wv, We, b2, s, n_total):
    Q4, W128 = emb4_s.shape
    pack = W128 // We.shape[0]
    Q = Q4 * pack
    VD = values_flat.shape[1]
    ED = We.shape[0]
    LD = Wv.shape[1]
    R = ROWS_BLK
    blk_off = s * (Q // R)
    grid = (Q // R,)

    common_in_specs = [
        pl.BlockSpec((R, VD), lambda i: (blk_off + i, 0)),
        pl.BlockSpec((R // pack, W128), lambda i: (i, 0)),
        pl.BlockSpec((VD, LD), lambda i: (0, 0)),
        pl.BlockSpec((ED, LD), lambda i: (0, 0)),
        pl.BlockSpec((1, LD), lambda i: (0, 0)),
    ]
    out_spec = pl.BlockSpec((R, LD), lambda i: (blk_off + i, 0))
    out_shape = jax.ShapeDtypeStruct((n_total, LD), jnp.float32)

    if prev is None:
        return pl.pallas_call(
            _mm_first_body,
            grid=grid,
            in_specs=common_in_specs,
            out_specs=out_spec,
            out_shape=out_shape,
        )(values_flat, emb4_s, Wv, We, b2)
    return pl.pallas_call(
        _mm_chain_body,
        grid=grid,
        in_specs=[pl.BlockSpec(memory_space=pl.ANY)] + common_in_specs,
        out_specs=out_spec,
        out_shape=out_shape,
        input_output_aliases={0: 0},
    )(prev, values_flat, emb4_s, Wv, We, b2)


def kernel(values, labels, emb_table, W, b):
    B, T, VD = values.shape
    ED = emb_table.shape[1]
    LD = W.shape[1]
    N = B * T
    Q = N // NSLICES

    labels_flat = labels.reshape(N).astype(jnp.int32)
    values_flat = values.reshape(N, VD)
    Wv = W[:VD]
    We = W[VD:VD + ED]
    b2 = b.reshape(1, LD)

    n_chunks = Q // (NW * CHUNK)
    embs = []
    for s in range(NSLICES):
        labels3 = lax.dynamic_slice_in_dim(labels_flat, s * Q, Q).reshape(
            NW, n_chunks, CHUNK)
        embs.append(_sc_gather(labels3, emb_table, n_chunks,
                               emb_table.shape[0], ED))

    out = None
    for s in range(NSLICES):
        out = _tc_project_slice(out, values_flat, embs[s], Wv, We, b2, s, N)
    return out.reshape(B, T, LD)


# speedup vs baseline: 1.7573x; 1.2256x over previous
import functools

import jax
import jax.numpy as jnp
from jax import lax
from jax.experimental import pallas as pl
from jax.experimental.pallas import tpu as pltpu
from jax.experimental.pallas import tpu_sc as plsc

NC, NS = 2, 16
NW = NC * NS
CHUNK = 128
NSLICES = 5
ROWS_BLK = 4096


@functools.partial(jax.jit, static_argnums=(2, 3, 4))
def _sc_gather(labels3, emb_table, n_chunks, n_rows, emb_d):
    mesh = plsc.VectorSubcoreMesh(
        core_axis_name="c", subcore_axis_name="s",
        num_cores=NC, num_subcores=NS,
    )
    L = 8
    G = 4
    assert n_chunks > L
    pack = 128 // emb_d
    assert n_chunks % pack == 0
    lines_w = n_chunks * CHUNK // pack

    @functools.partial(
        pl.kernel,
        out_type=jax.ShapeDtypeStruct(
            (NW * lines_w, pack * emb_d), jnp.float32),
        mesh=mesh,
        scratch_types=[
            pltpu.VMEM((n_chunks, CHUNK), jnp.int32),
            pltpu.VMEM((L, CHUNK, emb_d), jnp.float32),
            pltpu.SemaphoreType.DMA((L,)),
            pltpu.SemaphoreType.DMA((L,)),
        ],
        compiler_params=pltpu.CompilerParams(use_tc_tiling_on_sc=False),
    )
    def gather_kernel(labels_hbm, table2d, out_hbm, idx_v, rows_v, gsem, wsem):
        wid = lax.axis_index("c") * NS + lax.axis_index("s")
        base = wid * lines_w
        pltpu.sync_copy(labels_hbm.at[wid], idx_v)

        for k in range(G):
            pltpu.async_copy(
                table2d.at[idx_v.at[k]], rows_v.at[k % L], gsem.at[k % L])

        def body(j, carry):
            slot = lax.rem(j, L)
            pltpu.make_async_copy(
                table2d.at[idx_v.at[j]], rows_v.at[slot], gsem.at[slot]).wait()
            line0 = base + lax.div(j, pack) * CHUNK
            lane0 = lax.rem(j, pack) * emb_d
            dst = out_hbm.at[pl.ds(line0, CHUNK), pl.ds(lane0, emb_d)]
            pltpu.async_copy(rows_v.at[slot], dst, wsem.at[slot])

            nslot = lax.rem(j + G, L)

            @pl.when(j + G < n_chunks)
            def _issue_next():
                @pl.when(j + G >= L)
                def _drain_write():
                    pltpu.make_async_copy(
                        rows_v.at[nslot],
                        out_hbm.at[pl.ds(base, CHUNK),
                                   pl.ds(0, emb_d)],
                        wsem.at[nslot]).wait()

                pltpu.async_copy(
                    table2d.at[idx_v.at[j + G]], rows_v.at[nslot],
                    gsem.at[nslot])

            return carry

        lax.fori_loop(0, n_chunks, body, 0)

        for j in range(max(0, n_chunks - L), n_chunks):
            pltpu.make_async_copy(
                rows_v.at[j % L],
                out_hbm.at[pl.ds(base, CHUNK), pl.ds(0, emb_d)],
                wsem.at[j % L]).wait()

    return gather_kernel(labels3, emb_table)


def _mm_first_body(v_ref, e4_ref, wv_ref, we_ref, b_ref, o_ref):
    R = v_ref.shape[0]
    ED = we_ref.shape[0]
    e4 = e4_ref[...]
    pack = e4.shape[1] // ED
    n_sub = R // (pack * CHUNK)
    cs = [
        jnp.dot(e4[:, k * ED:(k + 1) * ED], we_ref[...],
                preferred_element_type=jnp.float32)
        for k in range(pack)
    ]
    pieces = [
        cs[k][sb * CHUNK:(sb + 1) * CHUNK]
        for sb in range(n_sub)
        for k in range(pack)
    ]
    e_contrib = jnp.concatenate(pieces, axis=0)
    acc = jnp.dot(v_ref[...], wv_ref[...], preferred_element_type=jnp.float32)
    o_ref[...] = acc + e_contrib + b_ref[...]


def _mm_chain_body(prev_ref, v_ref, e4_ref, wv_ref, we_ref, b_ref, o_ref):
    del prev_ref
    _mm_first_body(v_ref, e4_ref, wv_ref, we_ref, b_ref, o_ref)


@functools.partial(jax.jit, static_argnums=(6, 7))
def _tc_project_slice(prev, values_flat, emb4_s, Wv, We, b2, s, n_total):
    Q4, W128 = emb4_s.shape
    pack = W128 // We.shape[0]
    Q = Q4 * pack
    VD = values_flat.shape[1]
    ED = We.shape[0]
    LD = Wv.shape[1]
    R = ROWS_BLK
    blk_off = s * (Q // R)
    grid = (Q // R,)

    common_in_specs = [
        pl.BlockSpec((R, VD), lambda i: (blk_off + i, 0)),
        pl.BlockSpec((R // pack, W128), lambda i: (i, 0)),
        pl.BlockSpec((VD, LD), lambda i: (0, 0)),
        pl.BlockSpec((ED, LD), lambda i: (0, 0)),
        pl.BlockSpec((1, LD), lambda i: (0, 0)),
    ]
    out_spec = pl.BlockSpec((R, LD), lambda i: (blk_off + i, 0))
    out_shape = jax.ShapeDtypeStruct((n_total, LD), jnp.float32)

    if prev is None:
        return pl.pallas_call(
            _mm_first_body,
            grid=grid,
            in_specs=common_in_specs,
            out_specs=out_spec,
            out_shape=out_shape,
        )(values_flat, emb4_s, Wv, We, b2)
    return pl.pallas_call(
        _mm_chain_body,
        grid=grid,
        in_specs=[pl.BlockSpec(memory_space=pl.ANY)] + common_in_specs,
        out_specs=out_spec,
        out_shape=out_shape,
        input_output_aliases={0: 0},
    )(prev, values_flat, emb4_s, Wv, We, b2)


def kernel(values, labels, emb_table, W, b):
    B, T, VD = values.shape
    ED = emb_table.shape[1]
    LD = W.shape[1]
    N = B * T
    Q = N // NSLICES

    labels_flat = labels.reshape(N).astype(jnp.int32)
    values_flat = values.reshape(N, VD)
    Wv = W[:VD]
    We = W[VD:VD + ED]
    b2 = b.reshape(1, LD)

    n_chunks = Q // (NW * CHUNK)
    embs = []
    for s in range(NSLICES):
        labels3 = lax.dynamic_slice_in_dim(labels_flat, s * Q, Q).reshape(
            NW, n_chunks, CHUNK)
        embs.append(_sc_gather(labels3, emb_table, n_chunks,
                               emb_table.shape[0], ED))

    out = None
    for s in range(NSLICES):
        out = _tc_project_slice(out, values_flat, embs[s], Wv, We, b2, s, N)
    return out.reshape(B, T, LD)


# speedup vs baseline: 1.9182x; 1.0916x over previous
import functools

import jax
import jax.numpy as jnp
from jax import lax
from jax.experimental import pallas as pl
from jax.experimental.pallas import tpu as pltpu
from jax.experimental.pallas import tpu_sc as plsc

NC, NS = 2, 16
NW = NC * NS
CHUNK = 128
NSLICES = 5
ROWS_BLK = 8192


@functools.partial(jax.jit, static_argnums=(2, 3, 4))
def _sc_gather(labels3, emb_table, n_chunks, n_rows, emb_d):
    mesh = plsc.VectorSubcoreMesh(
        core_axis_name="c", subcore_axis_name="s",
        num_cores=NC, num_subcores=NS,
    )
    L = 8
    G = 4
    assert n_chunks > L
    pack = 128 // emb_d
    assert n_chunks % pack == 0
    lines_w = n_chunks * CHUNK // pack

    @functools.partial(
        pl.kernel,
        out_type=jax.ShapeDtypeStruct(
            (NW * lines_w, pack * emb_d), jnp.float32),
        mesh=mesh,
        scratch_types=[
            pltpu.VMEM((n_chunks, CHUNK), jnp.int32),
            pltpu.VMEM((L, CHUNK, emb_d), jnp.float32),
            pltpu.SemaphoreType.DMA((L,)),
            pltpu.SemaphoreType.DMA((L,)),
        ],
        compiler_params=pltpu.CompilerParams(use_tc_tiling_on_sc=False),
    )
    def gather_kernel(labels_hbm, table2d, out_hbm, idx_v, rows_v, gsem, wsem):
        wid = lax.axis_index("c") * NS + lax.axis_index("s")
        base = wid * lines_w
        pltpu.sync_copy(labels_hbm.at[wid], idx_v)

        for k in range(G):
            pltpu.async_copy(
                table2d.at[idx_v.at[k]], rows_v.at[k % L], gsem.at[k % L])

        def body(j, carry):
            slot = lax.rem(j, L)
            pltpu.make_async_copy(
                table2d.at[idx_v.at[j]], rows_v.at[slot], gsem.at[slot]).wait()
            line0 = base + lax.div(j, pack) * CHUNK
            lane0 = lax.rem(j, pack) * emb_d
            dst = out_hbm.at[pl.ds(line0, CHUNK), pl.ds(lane0, emb_d)]
            pltpu.async_copy(rows_v.at[slot], dst, wsem.at[slot])

            nslot = lax.rem(j + G, L)

            @pl.when(j + G < n_chunks)
            def _issue_next():
                @pl.when(j + G >= L)
                def _drain_write():
                    pltpu.make_async_copy(
                        rows_v.at[nslot],
                        out_hbm.at[pl.ds(base, CHUNK),
                                   pl.ds(0, emb_d)],
                        wsem.at[nslot]).wait()

                pltpu.async_copy(
                    table2d.at[idx_v.at[j + G]], rows_v.at[nslot],
                    gsem.at[nslot])

            return carry

        lax.fori_loop(0, n_chunks, body, 0)

        for j in range(max(0, n_chunks - L), n_chunks):
            pltpu.make_async_copy(
                rows_v.at[j % L],
                out_hbm.at[pl.ds(base, CHUNK), pl.ds(0, emb_d)],
                wsem.at[j % L]).wait()

    return gather_kernel(labels3, emb_table)


def _mm_first_body(v_ref, e4_ref, wv_ref, we_ref, b_ref, o_ref):
    R = v_ref.shape[0]
    ED = we_ref.shape[0]
    e4 = e4_ref[...]
    pack = e4.shape[1] // ED
    n_sub = R // (pack * CHUNK)
    cs = [
        jnp.dot(e4[:, k * ED:(k + 1) * ED], we_ref[...],
                preferred_element_type=jnp.float32)
        for k in range(pack)
    ]
    pieces = [
        cs[k][sb * CHUNK:(sb + 1) * CHUNK]
        for sb in range(n_sub)
        for k in range(pack)
    ]
    e_contrib = jnp.concatenate(pieces, axis=0)
    acc = jnp.dot(v_ref[...], wv_ref[...], preferred_element_type=jnp.float32)
    o_ref[...] = acc + e_contrib + b_ref[...]


def _mm_chain_body(prev_ref, v_ref, e4_ref, wv_ref, we_ref, b_ref, o_ref):
    del prev_ref
    _mm_first_body(v_ref, e4_ref, wv_ref, we_ref, b_ref, o_ref)


@functools.partial(jax.jit, static_argnums=(6, 7))
def _tc_project_slice(prev, values_flat, emb4_s, Wv, We, b2, s, n_total):
    Q4, W128 = emb4_s.shape
    pack = W128 // We.shape[0]
    Q = Q4 * pack
    VD = values_flat.shape[1]
    ED = We.shape[0]
    LD = Wv.shape[1]
    R = ROWS_BLK
    blk_off = s * (Q // R)
    grid = (Q // R,)

    common_in_specs = [
        pl.BlockSpec((R, VD), lambda i: (blk_off + i, 0)),
        pl.BlockSpec((R // pack, W128), lambda i: (i, 0)),
        pl.BlockSpec((VD, LD), lambda i: (0, 0)),
        pl.BlockSpec((ED, LD), lambda i: (0, 0)),
        pl.BlockSpec((1, LD), lambda i: (0, 0)),
    ]
    out_spec = pl.BlockSpec((R, LD), lambda i: (blk_off + i, 0))
    out_shape = jax.ShapeDtypeStruct((n_total, LD), jnp.float32)

    if prev is None:
        return pl.pallas_call(
            _mm_first_body,
            grid=grid,
            in_specs=common_in_specs,
            out_specs=out_spec,
            out_shape=out_shape,
        )(values_flat, emb4_s, Wv, We, b2)
    return pl.pallas_call(
        _mm_chain_body,
        grid=grid,
        in_specs=[pl.BlockSpec(memory_space=pl.ANY)] + common_in_specs,
        out_specs=out_spec,
        out_shape=out_shape,
        input_output_aliases={0: 0},
    )(prev, values_flat, emb4_s, Wv, We, b2)


def kernel(values, labels, emb_table, W, b):
    B, T, VD = values.shape
    ED = emb_table.shape[1]
    LD = W.shape[1]
    N = B * T
    Q = N // NSLICES

    labels_flat = labels.reshape(N).astype(jnp.int32)
    values_flat = values.reshape(N, VD)
    Wv = W[:VD]
    We = W[VD:VD + ED]
    b2 = b.reshape(1, LD)

    n_chunks = Q // (NW * CHUNK)
    embs = []
    for s in range(NSLICES):
        labels3 = lax.dynamic_slice_in_dim(labels_flat, s * Q, Q).reshape(
            NW, n_chunks, CHUNK)
        embs.append(_sc_gather(labels3, emb_table, n_chunks,
                               emb_table.shape[0], ED))

    out = None
    for s in range(NSLICES):
        out = _tc_project_slice(out, values_flat, embs[s], Wv, We, b2, s, N)
    return out.reshape(B, T, LD)


# speedup vs baseline: 1.9541x; 1.0187x over previous
import functools

import jax
import jax.numpy as jnp
from jax import lax
from jax.experimental import pallas as pl
from jax.experimental.pallas import tpu as pltpu
from jax.experimental.pallas import tpu_sc as plsc

NC, NS = 2, 16
NW = NC * NS
CHUNK = 128
NSLICES = 5
ROWS_BLK = 16384


@functools.partial(jax.jit, static_argnums=(2, 3, 4))
def _sc_gather(labels3, emb_table, n_chunks, n_rows, emb_d):
    mesh = plsc.VectorSubcoreMesh(
        core_axis_name="c", subcore_axis_name="s",
        num_cores=NC, num_subcores=NS,
    )
    L = 8
    G = 4
    assert n_chunks > L
    pack = 128 // emb_d
    assert n_chunks % pack == 0
    lines_w = n_chunks * CHUNK // pack

    @functools.partial(
        pl.kernel,
        out_type=jax.ShapeDtypeStruct(
            (NW * lines_w, pack * emb_d), jnp.float32),
        mesh=mesh,
        scratch_types=[
            pltpu.VMEM((n_chunks, CHUNK), jnp.int32),
            pltpu.VMEM((L, CHUNK, emb_d), jnp.float32),
            pltpu.SemaphoreType.DMA((L,)),
            pltpu.SemaphoreType.DMA((L,)),
        ],
        compiler_params=pltpu.CompilerParams(use_tc_tiling_on_sc=False),
    )
    def gather_kernel(labels_hbm, table2d, out_hbm, idx_v, rows_v, gsem, wsem):
        wid = lax.axis_index("c") * NS + lax.axis_index("s")
        base = wid * lines_w
        pltpu.sync_copy(labels_hbm.at[wid], idx_v)

        for k in range(G):
            pltpu.async_copy(
                table2d.at[idx_v.at[k]], rows_v.at[k % L], gsem.at[k % L])

        def body(j, carry):
            slot = lax.rem(j, L)
            pltpu.make_async_copy(
                table2d.at[idx_v.at[j]], rows_v.at[slot], gsem.at[slot]).wait()
            line0 = base + lax.div(j, pack) * CHUNK
            lane0 = lax.rem(j, pack) * emb_d
            dst = out_hbm.at[pl.ds(line0, CHUNK), pl.ds(lane0, emb_d)]
            pltpu.async_copy(rows_v.at[slot], dst, wsem.at[slot])

            nslot = lax.rem(j + G, L)

            @pl.when(j + G < n_chunks)
            def _issue_next():
                @pl.when(j + G >= L)
                def _drain_write():
                    pltpu.make_async_copy(
                        rows_v.at[nslot],
                        out_hbm.at[pl.ds(base, CHUNK),
                                   pl.ds(0, emb_d)],
                        wsem.at[nslot]).wait()

                pltpu.async_copy(
                    table2d.at[idx_v.at[j + G]], rows_v.at[nslot],
                    gsem.at[nslot])

            return carry

        lax.fori_loop(0, n_chunks, body, 0)

        for j in range(max(0, n_chunks - L), n_chunks):
            pltpu.make_async_copy(
                rows_v.at[j % L],
                out_hbm.at[pl.ds(base, CHUNK), pl.ds(0, emb_d)],
                wsem.at[j % L]).wait()

    return gather_kernel(labels3, emb_table)


def _mm_first_body(v_ref, e4_ref, wv_ref, we_ref, b_ref, o_ref):
    R = v_ref.shape[0]
    ED = we_ref.shape[0]
    e4 = e4_ref[...]
    pack = e4.shape[1] // ED
    n_sub = R // (pack * CHUNK)
    cs = [
        jnp.dot(e4[:, k * ED:(k + 1) * ED], we_ref[...],
                preferred_element_type=jnp.float32)
        for k in range(pack)
    ]
    pieces = [
        cs[k][sb * CHUNK:(sb + 1) * CHUNK]
        for sb in range(n_sub)
        for k in range(pack)
    ]
    e_contrib = jnp.concatenate(pieces, axis=0)
    acc = jnp.dot(v_ref[...], wv_ref[...], preferred_element_type=jnp.float32)
    o_ref[...] = acc + e_contrib + b_ref[...]


def _mm_chain_body(prev_ref, v_ref, e4_ref, wv_ref, we_ref, b_ref, o_ref):
    del prev_ref
    _mm_first_body(v_ref, e4_ref, wv_ref, we_ref, b_ref, o_ref)


@functools.partial(jax.jit, static_argnums=(6, 7))
def _tc_project_slice(prev, values_flat, emb4_s, Wv, We, b2, s, n_total):
    Q4, W128 = emb4_s.shape
    pack = W128 // We.shape[0]
    Q = Q4 * pack
    VD = values_flat.shape[1]
    ED = We.shape[0]
    LD = Wv.shape[1]
    R = ROWS_BLK
    blk_off = s * (Q // R)
    grid = (Q // R,)

    common_in_specs = [
        pl.BlockSpec((R, VD), lambda i: (blk_off + i, 0)),
        pl.BlockSpec((R // pack, W128), lambda i: (i, 0)),
        pl.BlockSpec((VD, LD), lambda i: (0, 0)),
        pl.BlockSpec((ED, LD), lambda i: (0, 0)),
        pl.BlockSpec((1, LD), lambda i: (0, 0)),
    ]
    out_spec = pl.BlockSpec((R, LD), lambda i: (blk_off + i, 0))
    out_shape = jax.ShapeDtypeStruct((n_total, LD), jnp.float32)

    if prev is None:
        return pl.pallas_call(
            _mm_first_body,
            grid=grid,
            in_specs=common_in_specs,
            out_specs=out_spec,
            out_shape=out_shape,
        )(values_flat, emb4_s, Wv, We, b2)
    return pl.pallas_call(
        _mm_chain_body,
        grid=grid,
        in_specs=[pl.BlockSpec(memory_space=pl.ANY)] + common_in_specs,
        out_specs=out_spec,
        out_shape=out_shape,
        input_output_aliases={0: 0},
    )(prev, values_flat, emb4_s, Wv, We, b2)


def kernel(values, labels, emb_table, W, b):
    B, T, VD = values.shape
    ED = emb_table.shape[1]
    LD = W.shape[1]
    N = B * T
    Q = N // NSLICES

    labels_flat = labels.reshape(N).astype(jnp.int32)
    values_flat = values.reshape(N, VD)
    Wv = W[:VD]
    We = W[VD:VD + ED]
    b2 = b.reshape(1, LD)

    n_chunks = Q // (NW * CHUNK)
    embs = []
    for s in range(NSLICES):
        labels3 = lax.dynamic_slice_in_dim(labels_flat, s * Q, Q).reshape(
            NW, n_chunks, CHUNK)
        embs.append(_sc_gather(labels3, emb_table, n_chunks,
                               emb_table.shape[0], ED))

    out = None
    for s in range(NSLICES):
        out = _tc_project_slice(out, values_flat, embs[s], Wv, We, b2, s, N)
    return out.reshape(B, T, LD)
